# baseline (speedup 1.0000x reference)
import functools

import jax
import jax.numpy as jnp
from jax import lax
from jax.experimental import pallas as pl
from jax.experimental.pallas import tpu as pltpu
from jax.experimental.pallas import tpu_sc as plsc

_EPS = 1e-12
_TOPK = 10
_SEG = 128
_BLOCK_K = 2048
_L = 16
_NW = 32


def _score_body(q_ref, kt_ref, s_ref, m_ref, qn_ref, *, n_keys, n_blocks):
    b = pl.program_id(0)
    Q = q_ref.shape[0]

    @pl.when(b == 0)
    def _init():
        q = q_ref[...]
        qnorm = jnp.sqrt(jnp.sum(q * q, axis=1, keepdims=True))
        qn_ref[...] = q / (qnorm + _EPS)

    kt = kt_ref[...]
    ss = jnp.sum(kt * kt, axis=0, keepdims=True)
    inv = 1.0 / (jnp.sqrt(ss) + _EPS)
    s = jnp.dot(qn_ref[...], kt * inv, preferred_element_type=jnp.float32)

    col = jax.lax.broadcasted_iota(jnp.int32, (1, _BLOCK_K), 1) + b * _BLOCK_K
    s = jnp.where(col < n_keys, s, -jnp.inf)
    s_ref[...] = s[None]
    m_ref[...] = jnp.max(s.reshape(Q, _BLOCK_K // _SEG, _SEG), axis=2)[None]


def _merge16(es, ei, x, ix):
    xs, ixs = plsc.sort_key_val(x, ix)
    w = es >= xs
    ms = jnp.where(w, es, xs)
    mi = jnp.where(w, ei, ixs)
    ns, ni = plsc.sort_key_val(ms, mi, descending=True)
    return ns, ni


def _sc_topk(m_hbm, sv_hbm, out_s, out_i,
             m_v, seg_v, idx_v, rows_v, os_v, oi_v, sem, *, n_seg, q_per_w,
             n_q):
    wid = lax.axis_index("s") * 2 + lax.axis_index("c")
    q0 = wid * q_per_w
    pltpu.sync_copy(m_hbm.at[pl.ds(q0 * n_seg, q_per_w * n_seg)], m_v)
    iota = jax.lax.broadcasted_iota(jnp.int32, (_L,), 0)
    neg = jnp.full((_L,), -jnp.inf, dtype=jnp.float32)
    zero_i = jnp.zeros((_L,), dtype=jnp.int32)
    n_vregs = n_seg // _L

    def per_query(qi, carry):
        qsplat = jnp.broadcast_to(qi, (_L,)).astype(jnp.int32)

        def scan_body(v, c):
            es, ei = c
            x = m_v[pl.ds(qi * n_seg + v * _L, _L)]
            thr = jnp.full((_L,), es[9], jnp.float32)
            hit = plsc.all_reduce_population_count(x > thr)[0]
            return lax.cond(
                hit > 0,
                lambda: _merge16(es, ei, x, v * _L + iota),
                lambda: (es, ei))

        es, ei = lax.fori_loop(0, n_vregs, scan_body, (neg, zero_i))

        seg_v[...] = ei
        spb = _BLOCK_K // _SEG
        idx_v[...] = ((ei // spb) * (n_q * spb) + (q0 + qi) * spb
                      + (ei % spb))
        pltpu.async_copy(sv_hbm.at[idx_v], rows_v, sem).wait()

        def ref_body(t, c):
            fs, fi = c
            i = t // (_SEG // _L)
            j = t % (_SEG // _L)
            isplat = jnp.broadcast_to(i, (_L,)).astype(jnp.int32)
            base = plsc.load_gather(seg_v, [isplat])
            row = rows_v[i, pl.ds(j * _L, _L)]
            gix = base * _SEG + j * _L + iota
            thr = jnp.full((_L,), fs[9], jnp.float32)
            hit = plsc.all_reduce_population_count(row > thr)[0]
            return lax.cond(
                hit > 0,
                lambda: _merge16(fs, fi, row, gix),
                lambda: (fs, fi))

        fs, fi = lax.fori_loop(0, _L * (_SEG // _L), ref_body, (neg, zero_i))
        os_v[pl.ds(qi * _L, _L)] = fs
        oi_v[pl.ds(qi * _L, _L)] = fi
        return carry

    lax.fori_loop(0, q_per_w, per_query, 0)
    pltpu.sync_copy(os_v, out_s.at[pl.ds(q0 * _L, q_per_w * _L)])
    pltpu.sync_copy(oi_v, out_i.at[pl.ds(q0 * _L, q_per_w * _L)])


def kernel(queries, keys, k):
    del k
    Q, D = queries.shape
    K = keys.shape[0]
    n_blocks = pl.cdiv(K, _BLOCK_K)
    kp = n_blocks * _BLOCK_K
    n_seg = kp // _SEG
    q_per_w = Q // _NW

    keys_t = jnp.pad(keys.T, ((0, 0), (0, kp - K)))

    body = functools.partial(_score_body, n_keys=K, n_blocks=n_blocks)
    s_pad, m = pl.pallas_call(
        body,
        grid=(n_blocks,),
        in_specs=[
            pl.BlockSpec((Q, D), lambda b: (0, 0)),
            pl.BlockSpec((D, _BLOCK_K), lambda b: (0, b)),
        ],
        out_specs=[
            pl.BlockSpec((1, Q, _BLOCK_K), lambda b: (b, 0, 0)),
            pl.BlockSpec((1, Q, _BLOCK_K // _SEG), lambda b: (b, 0, 0)),
        ],
        out_shape=[
            jax.ShapeDtypeStruct((n_blocks, Q, _BLOCK_K), jnp.float32),
            jax.ShapeDtypeStruct((n_blocks, Q, _BLOCK_K // _SEG), jnp.float32),
        ],
        scratch_shapes=[
            pltpu.VMEM((Q, D), jnp.float32),
        ],
    )(queries, keys_t)

    sv = s_pad.reshape(n_blocks * Q * (_BLOCK_K // _SEG), _SEG)
    m_flat = m.transpose(1, 0, 2).reshape(Q * n_seg)

    sc = functools.partial(_sc_topk, n_seg=n_seg, q_per_w=q_per_w, n_q=Q)
    out_s, out_i = pl.kernel(
        sc,
        out_type=[
            jax.ShapeDtypeStruct((Q * _L,), jnp.float32),
            jax.ShapeDtypeStruct((Q * _L,), jnp.int32),
        ],
        mesh=plsc.VectorSubcoreMesh(core_axis_name="c", subcore_axis_name="s"),
        compiler_params=pltpu.CompilerParams(needs_layout_passes=False),
        scratch_types=[
            pltpu.VMEM((q_per_w * n_seg,), jnp.float32),
            pltpu.VMEM((_L,), jnp.int32),
            pltpu.VMEM((_L,), jnp.int32),
            pltpu.VMEM((_L, _SEG), jnp.float32),
            pltpu.VMEM((q_per_w * _L,), jnp.float32),
            pltpu.VMEM((q_per_w * _L,), jnp.int32),
            pltpu.SemaphoreType.DMA,
        ],
    )(m_flat, sv)

    return (out_s.reshape(Q, _L)[:, :_TOPK],
            out_i.reshape(Q, _L)[:, :_TOPK])


# speedup vs baseline: 1.2232x; 1.2232x over previous
import functools

import jax
import jax.numpy as jnp
from jax import lax
from jax.experimental import pallas as pl
from jax.experimental.pallas import tpu as pltpu
from jax.experimental.pallas import tpu_sc as plsc

_EPS = 1e-12
_TOPK = 10
_SEG = 128
_BLOCK_K = 2048
_L = 16
_NW = 32


def _score_body(q_ref, kt_ref, s_ref, m_ref, qn_ref, *, n_keys, n_blocks):
    b = pl.program_id(0)
    Q = q_ref.shape[0]

    @pl.when(b == 0)
    def _init():
        q = q_ref[...]
        qnorm = jnp.sqrt(jnp.sum(q * q, axis=1, keepdims=True))
        qn_ref[...] = q / (qnorm + _EPS)

    kt = kt_ref[...]
    ss = jnp.sum(kt * kt, axis=0, keepdims=True)
    inv = 1.0 / (jnp.sqrt(ss) + _EPS)
    s = jnp.dot(qn_ref[...], kt * inv, preferred_element_type=jnp.float32)

    col = jax.lax.broadcasted_iota(jnp.int32, (1, _BLOCK_K), 1) + b * _BLOCK_K
    s = jnp.where(col < n_keys, s, -jnp.inf)
    s_ref[...] = s
    m_ref[...] = jnp.max(s.reshape(Q, _BLOCK_K // _SEG, _SEG), axis=2)[None]


def _merge16(es, ei, x, ix):
    xs, ixs = plsc.sort_key_val(x, ix)
    w = es >= xs
    ms = jnp.where(w, es, xs)
    mi = jnp.where(w, ei, ixs)
    ns, ni = plsc.sort_key_val(ms, mi, descending=True)
    return ns, ni


def _sc_topk(m_hbm, sv_hbm, out_s, out_i,
             m_v, seg_v0, idx_v0, rows_v0, seg_v1, idx_v1, rows_v1,
             os_v, oi_v, sem0, sem1, *, n_seg, q_per_w):
    wid = lax.axis_index("s") * 2 + lax.axis_index("c")
    q0 = wid * q_per_w
    pltpu.sync_copy(m_hbm.at[pl.ds(q0 * n_seg, q_per_w * n_seg)], m_v)
    iota = jax.lax.broadcasted_iota(jnp.int32, (_L,), 0)
    neg = jnp.full((_L,), -jnp.inf, dtype=jnp.float32)
    zero_i = jnp.zeros((_L,), dtype=jnp.int32)
    n_vregs = n_seg // _L

    def scan_query(qi):
        def scan_body(v, c):
            es, ei = c
            x = m_v[pl.ds(qi * n_seg + v * _L, _L)]
            thr = jnp.full((_L,), es[9], jnp.float32)
            hit = plsc.all_reduce_population_count(x > thr)[0]
            return lax.cond(
                hit > 0,
                lambda: _merge16(es, ei, x, v * _L + iota),
                lambda: (es, ei))

        return lax.fori_loop(0, n_vregs, scan_body, (neg, zero_i))

    def issue_gather(qi, ei, seg_ref, idx_ref, rows_ref, sem):
        seg_ref[...] = ei
        idx_ref[...] = (q0 + qi) * n_seg + ei
        return pltpu.async_copy(sv_hbm.at[idx_ref], rows_ref, sem)

    def refine(qi, seg_ref, rows_ref):
        def ref_body(t, c):
            fs, fi = c
            i = t // (_SEG // _L)
            j = t % (_SEG // _L)
            isplat = jnp.broadcast_to(i, (_L,)).astype(jnp.int32)
            base = plsc.load_gather(seg_ref, [isplat])
            row = rows_ref[i, pl.ds(j * _L, _L)]
            gix = base * _SEG + j * _L + iota
            thr = jnp.full((_L,), fs[9], jnp.float32)
            hit = plsc.all_reduce_population_count(row > thr)[0]
            return lax.cond(
                hit > 0.5,
                lambda: _merge16(fs, fi, row, gix),
                lambda: (fs, fi))

        fs, fi = lax.fori_loop(0, _L * (_SEG // _L), ref_body, (neg, zero_i))
        os_v[pl.ds(qi * _L, _L)] = fs
        oi_v[pl.ds(qi * _L, _L)] = fi

    def wait_gather(idx_ref, rows_ref, sem):
        pltpu.make_async_copy(sv_hbm.at[idx_ref], rows_ref, sem).wait()

    def per_pair(t, carry):
        qa = 2 * t
        qb = 2 * t + 1
        _, ei0 = scan_query(qa)
        issue_gather(qa, ei0, seg_v0, idx_v0, rows_v0, sem0)

        @pl.when(t > 0)
        def _refine_prev_odd():
            wait_gather(idx_v1, rows_v1, sem1)
            refine(qa - 1, seg_v1, rows_v1)

        _, ei1 = scan_query(qb)
        issue_gather(qb, ei1, seg_v1, idx_v1, rows_v1, sem1)
        wait_gather(idx_v0, rows_v0, sem0)
        refine(qa, seg_v0, rows_v0)
        return carry

    lax.fori_loop(0, q_per_w // 2, per_pair, 0)
    wait_gather(idx_v1, rows_v1, sem1)
    refine(q_per_w - 1, seg_v1, rows_v1)
    pltpu.sync_copy(os_v, out_s.at[pl.ds(q0 * _L, q_per_w * _L)])
    pltpu.sync_copy(oi_v, out_i.at[pl.ds(q0 * _L, q_per_w * _L)])


def kernel(queries, keys, k):
    del k
    Q, D = queries.shape
    K = keys.shape[0]
    n_blocks = pl.cdiv(K, _BLOCK_K)
    kp = n_blocks * _BLOCK_K
    n_seg = kp // _SEG
    q_per_w = Q // _NW

    keys_t = jnp.pad(keys.T, ((0, 0), (0, kp - K)))

    body = functools.partial(_score_body, n_keys=K, n_blocks=n_blocks)
    s_pad, m = pl.pallas_call(
        body,
        grid=(n_blocks,),
        in_specs=[
            pl.BlockSpec((Q, D), lambda b: (0, 0)),
            pl.BlockSpec((D, _BLOCK_K), lambda b: (0, b)),
        ],
        out_specs=[
            pl.BlockSpec((Q, _BLOCK_K), lambda b: (0, b)),
            pl.BlockSpec((1, Q, _BLOCK_K // _SEG), lambda b: (b, 0, 0)),
        ],
        out_shape=[
            jax.ShapeDtypeStruct((Q, kp), jnp.float32),
            jax.ShapeDtypeStruct((n_blocks, Q, _BLOCK_K // _SEG), jnp.float32),
        ],
        scratch_shapes=[
            pltpu.VMEM((Q, D), jnp.float32),
        ],
    )(queries, keys_t)

    sv = s_pad.reshape(Q * n_seg, _SEG)
    m_flat = m.transpose(1, 0, 2).reshape(Q * n_seg)

    sc = functools.partial(_sc_topk, n_seg=n_seg, q_per_w=q_per_w)
    out_s, out_i = pl.kernel(
        sc,
        out_type=[
            jax.ShapeDtypeStruct((Q * _L,), jnp.float32),
            jax.ShapeDtypeStruct((Q * _L,), jnp.int32),
        ],
        mesh=plsc.VectorSubcoreMesh(core_axis_name="c", subcore_axis_name="s"),
        compiler_params=pltpu.CompilerParams(needs_layout_passes=False),
        scratch_types=[
            pltpu.VMEM((q_per_w * n_seg,), jnp.float32),
            pltpu.VMEM((_L,), jnp.int32),
            pltpu.VMEM((_L,), jnp.int32),
            pltpu.VMEM((_L, _SEG), jnp.float32),
            pltpu.VMEM((_L,), jnp.int32),
            pltpu.VMEM((_L,), jnp.int32),
            pltpu.VMEM((_L, _SEG), jnp.float32),
            pltpu.VMEM((q_per_w * _L,), jnp.float32),
            pltpu.VMEM((q_per_w * _L,), jnp.int32),
            pltpu.SemaphoreType.DMA,
            pltpu.SemaphoreType.DMA,
        ],
    )(m_flat, sv)

    return (out_s.reshape(Q, _L)[:, :_TOPK],
            out_i.reshape(Q, _L)[:, :_TOPK])


# speedup vs baseline: 1.3016x; 1.0641x over previous
import functools

import jax
import jax.numpy as jnp
from jax import lax
from jax.experimental import pallas as pl
from jax.experimental.pallas import tpu as pltpu
from jax.experimental.pallas import tpu_sc as plsc

_EPS = 1e-12
_TOPK = 10
_SEG = 128
_BLOCK_K = 2048
_L = 16
_NW = 32


def _score_body(q_ref, kt_ref, s_ref, m_ref, qn_ref, *, n_keys, n_blocks):
    b = pl.program_id(0)
    Q = q_ref.shape[0]

    @pl.when(b == 0)
    def _init():
        q = q_ref[...]
        qnorm = jnp.sqrt(jnp.sum(q * q, axis=1, keepdims=True))
        qn_ref[...] = q / (qnorm + _EPS)

    kt = kt_ref[...]
    ss = jnp.sum(kt * kt, axis=0, keepdims=True)
    inv = 1.0 / (jnp.sqrt(ss) + _EPS)
    s = jnp.dot(qn_ref[...], kt * inv, preferred_element_type=jnp.float32)

    col = jax.lax.broadcasted_iota(jnp.int32, (1, _BLOCK_K), 1) + b * _BLOCK_K
    s = jnp.where(col < n_keys, s, -jnp.inf)
    s_ref[...] = s
    m_ref[...] = jnp.max(s.reshape(Q, _BLOCK_K // _SEG, _SEG), axis=2)[None]


def _merge16(es, ei, x, ix):
    xs, ixs = plsc.sort_key_val(x, ix)
    w = es >= xs
    ms = jnp.where(w, es, xs)
    mi = jnp.where(w, ei, ixs)
    ns, ni = plsc.sort_key_val(ms, mi, descending=True)
    return ns, ni


def _sc_topk(m_hbm, sv_hbm, out_s, out_i,
             m_v, seg_v0, idx_v0, rows_v0, seg_v1, idx_v1, rows_v1,
             os_v, oi_v, sem0, sem1, *, n_seg, q_per_w):
    wid = lax.axis_index("s") * 2 + lax.axis_index("c")
    q0 = wid * q_per_w
    pltpu.sync_copy(m_hbm.at[pl.ds(q0 * n_seg, q_per_w * n_seg)], m_v)
    iota = jax.lax.broadcasted_iota(jnp.int32, (_L,), 0)
    neg = jnp.full((_L,), -jnp.inf, dtype=jnp.float32)
    zero_i = jnp.zeros((_L,), dtype=jnp.int32)
    n_vregs = n_seg // _L

    def scan_query(qi):
        def scan_body(v, c):
            es, ei = c
            x = m_v[pl.ds(qi * n_seg + v * _L, _L)]
            thr = jnp.full((_L,), es[9], jnp.float32)
            hit = plsc.all_reduce_population_count(x > thr)[0]
            return lax.cond(
                hit > 0,
                lambda: _merge16(es, ei, x, v * _L + iota),
                lambda: (es, ei))

        return lax.fori_loop(0, n_vregs, scan_body, (neg, zero_i))

    def issue_gather(qi, ei, seg_ref, idx_ref, rows_ref, sem):
        seg_ref[...] = ei
        idx_ref[...] = (q0 + qi) * n_seg + ei
        return pltpu.async_copy(sv_hbm.at[idx_ref], rows_ref, sem)

    def refine(qi, seg_ref, rows_ref, es):
        def seg_active(c):
            i, fs, fi = c
            thr = jnp.full((_L,), fs[9], jnp.float32)
            n_act = plsc.all_reduce_population_count(es > thr)[0]
            return jnp.logical_and(i < _L, n_act > i)

        def seg_body(c):
            i, fs, fi = c
            isplat = jnp.broadcast_to(i, (_L,)).astype(jnp.int32)
            base = plsc.load_gather(seg_ref, [isplat])

            def vreg_body(j, c2):
                fs2, fi2 = c2
                row = rows_ref[i, pl.ds(j * _L, _L)]
                gix = base * _SEG + j * _L + iota
                thr = jnp.full((_L,), fs2[9], jnp.float32)
                hit = plsc.all_reduce_population_count(row > thr)[0]
                return lax.cond(
                    hit > 0,
                    lambda: _merge16(fs2, fi2, row, gix),
                    lambda: (fs2, fi2))

            fs, fi = lax.fori_loop(0, _SEG // _L, vreg_body, (fs, fi))
            return i + 1, fs, fi

        _, fs, fi = lax.while_loop(seg_active, seg_body, (0, neg, zero_i))
        os_v[pl.ds(qi * _L, _L)] = fs
        oi_v[pl.ds(qi * _L, _L)] = fi

    def wait_gather(idx_ref, rows_ref, sem):
        pltpu.make_async_copy(sv_hbm.at[idx_ref], rows_ref, sem).wait()

    def per_pair(t, es1_prev):
        qa = 2 * t
        qb = 2 * t + 1
        es0, ei0 = scan_query(qa)
        issue_gather(qa, ei0, seg_v0, idx_v0, rows_v0, sem0)

        @pl.when(t > 0)
        def _refine_prev_odd():
            wait_gather(idx_v1, rows_v1, sem1)
            refine(qa - 1, seg_v1, rows_v1, es1_prev)

        es1, ei1 = scan_query(qb)
        issue_gather(qb, ei1, seg_v1, idx_v1, rows_v1, sem1)
        wait_gather(idx_v0, rows_v0, sem0)
        refine(qa, seg_v0, rows_v0, es0)
        return es1

    es1_last = lax.fori_loop(0, q_per_w // 2, per_pair, neg)
    wait_gather(idx_v1, rows_v1, sem1)
    refine(q_per_w - 1, seg_v1, rows_v1, es1_last)
    pltpu.sync_copy(os_v, out_s.at[pl.ds(q0 * _L, q_per_w * _L)])
    pltpu.sync_copy(oi_v, out_i.at[pl.ds(q0 * _L, q_per_w * _L)])


def kernel(queries, keys, k):
    del k
    Q, D = queries.shape
    K = keys.shape[0]
    n_blocks = pl.cdiv(K, _BLOCK_K)
    kp = n_blocks * _BLOCK_K
    n_seg = kp // _SEG
    q_per_w = Q // _NW

    keys_t = jnp.pad(keys.T, ((0, 0), (0, kp - K)))

    body = functools.partial(_score_body, n_keys=K, n_blocks=n_blocks)
    s_pad, m = pl.pallas_call(
        body,
        grid=(n_blocks,),
        in_specs=[
            pl.BlockSpec((Q, D), lambda b: (0, 0)),
            pl.BlockSpec((D, _BLOCK_K), lambda b: (0, b)),
        ],
        out_specs=[
            pl.BlockSpec((Q, _BLOCK_K), lambda b: (0, b)),
            pl.BlockSpec((1, Q, _BLOCK_K // _SEG), lambda b: (b, 0, 0)),
        ],
        out_shape=[
            jax.ShapeDtypeStruct((Q, kp), jnp.float32),
            jax.ShapeDtypeStruct((n_blocks, Q, _BLOCK_K // _SEG), jnp.float32),
        ],
        scratch_shapes=[
            pltpu.VMEM((Q, D), jnp.float32),
        ],
    )(queries, keys_t)

    sv = s_pad.reshape(Q * n_seg, _SEG)
    m_flat = m.transpose(1, 0, 2).reshape(Q * n_seg)

    sc = functools.partial(_sc_topk, n_seg=n_seg, q_per_w=q_per_w)
    out_s, out_i = pl.kernel(
        sc,
        out_type=[
            jax.ShapeDtypeStruct((Q * _L,), jnp.float32),
            jax.ShapeDtypeStruct((Q * _L,), jnp.int32),
        ],
        mesh=plsc.VectorSubcoreMesh(core_axis_name="c", subcore_axis_name="s"),
        compiler_params=pltpu.CompilerParams(needs_layout_passes=False),
        scratch_types=[
            pltpu.VMEM((q_per_w * n_seg,), jnp.float32),
            pltpu.VMEM((_L,), jnp.int32),
            pltpu.VMEM((_L,), jnp.int32),
            pltpu.VMEM((_L, _SEG), jnp.float32),
            pltpu.VMEM((_L,), jnp.int32),
            pltpu.VMEM((_L,), jnp.int32),
            pltpu.VMEM((_L, _SEG), jnp.float32),
            pltpu.VMEM((q_per_w * _L,), jnp.float32),
            pltpu.VMEM((q_per_w * _L,), jnp.int32),
            pltpu.SemaphoreType.DMA,
            pltpu.SemaphoreType.DMA,
        ],
    )(m_flat, sv)

    return (out_s.reshape(Q, _L)[:, :_TOPK],
            out_i.reshape(Q, _L)[:, :_TOPK])
